# baseline (speedup 1.0000x reference)
import functools

import jax
import jax.numpy as jnp
from jax import lax
from jax.experimental import pallas as pl
from jax.experimental.pallas import tpu as pltpu
from jax.experimental.pallas import tpu_sc as plsc

N_NODES = 10000
K_SAMPLE = 32
D_FEAT = 128
HIDDEN = 32
OUT_U = 16
N_HEADS = 4

CHUNK = 128
N_CHUNKS = (N_NODES * K_SAMPLE) // CHUNK

PACK = 4
ROWS_PER_NODE = K_SAMPLE // PACK


def _precompute_body(x_ref, w1_ref, b1_ref, y_ref, c_ref):
    x = x_ref[...]
    y_ref[...] = jnp.dot(x, w1_ref[0:D_FEAT, :],
                         preferred_element_type=jnp.float32)
    c_ref[...] = jnp.dot(x, w1_ref[D_FEAT:2 * D_FEAT, :],
                         preferred_element_type=jnp.float32) + b1_ref[...]


def _precompute(x, W1, b1):
    return pl.pallas_call(
        _precompute_body,
        out_shape=(
            jax.ShapeDtypeStruct((N_NODES, HIDDEN), jnp.float32),
            jax.ShapeDtypeStruct((N_NODES, HIDDEN), jnp.float32),
        ),
    )(x, W1, b1)


def _sc_gather(y, idx2d):
    info = plsc.get_sparse_core_info()
    nc, ns = info.num_cores, info.num_subcores
    nw = nc * ns
    per_w = -(-N_CHUNKS // nw) + (-(-N_CHUNKS // nw)) % 2
    padded = nw * per_w

    mesh = plsc.VectorSubcoreMesh(core_axis_name="c", subcore_axis_name="s")

    @functools.partial(
        pl.kernel,
        mesh=mesh,
        compiler_params=pltpu.CompilerParams(use_tc_tiling_on_sc=False),
        out_type=jax.ShapeDtypeStruct((N_CHUNKS, CHUNK, HIDDEN),
                                      jnp.float32),
        scratch_types=[
            pltpu.VMEM((per_w, CHUNK), jnp.int32),
            pltpu.VMEM((CHUNK, HIDDEN), jnp.float32),
            pltpu.VMEM((CHUNK, HIDDEN), jnp.float32),
            pltpu.SemaphoreType.DMA,
            pltpu.SemaphoreType.DMA,
            pltpu.SemaphoreType.DMA,
            pltpu.SemaphoreType.DMA,
        ],
    )
    def k(y_hbm, idx_hbm, out_hbm, idx_v, rows0, rows1, g0, g1, w0, w1):
        wid = lax.axis_index("s") * nc + lax.axis_index("c")
        start = wid * per_w
        pltpu.sync_copy(idx_hbm.at[pl.ds(start, per_w)], idx_v)
        pltpu.async_copy(y_hbm.at[idx_v.at[0]], rows0, g0)
        pltpu.async_copy(y_hbm.at[idx_v.at[1]], rows1, g1)

        def body(i, _):
            c0, c1 = 2 * i, 2 * i + 1
            go0, go1 = start + c0, start + c1
            pltpu.make_async_copy(y_hbm.at[idx_v.at[c0]], rows0, g0).wait()

            @pl.when(go0 < N_CHUNKS)
            def _():
                pltpu.async_copy(rows0, out_hbm.at[go0], w0)

            pltpu.make_async_copy(y_hbm.at[idx_v.at[c1]], rows1, g1).wait()

            @pl.when(go1 < N_CHUNKS)
            def _():
                pltpu.async_copy(rows1, out_hbm.at[go1], w1)

            @pl.when(i < per_w // 2 - 1)
            def _():
                @pl.when(go0 < N_CHUNKS)
                def _():
                    pltpu.make_async_copy(rows0, out_hbm.at[go0], w0).wait()

                pltpu.async_copy(y_hbm.at[idx_v.at[c0 + 2]], rows0, g0)

                @pl.when(go1 < N_CHUNKS)
                def _():
                    pltpu.make_async_copy(rows1, out_hbm.at[go1], w1).wait()

                pltpu.async_copy(y_hbm.at[idx_v.at[c1 + 2]], rows1, g1)

            return ()

        lax.fori_loop(0, per_w // 2, body, ())
        last0, last1 = start + per_w - 2, start + per_w - 1

        @pl.when(last0 < N_CHUNKS)
        def _():
            pltpu.make_async_copy(rows0, out_hbm.at[last0], w0).wait()

        @pl.when(last1 < N_CHUNKS)
        def _():
            pltpu.make_async_copy(rows1, out_hbm.at[last1], w1).wait()

    idx_pad = jnp.zeros((padded, CHUNK), jnp.int32).at[:N_CHUNKS].set(idx2d)
    return k(y, idx_pad)


def _dense_body(g_ref, c_ref, w2bd_ref, b2bd_ref, wabd_ref, babd_ref,
                out_ref, *, blk):
    B, K, H, U = blk, K_SAMPLE, N_HEADS, OUT_U
    R = B * ROWS_PER_NODE
    f32 = jnp.float32

    g = g_ref[...]
    c = c_ref[...]
    c_rep = jnp.concatenate([c, c, c, c], axis=1)
    c_rep = jnp.broadcast_to(c_rep[:, None, :],
                             (B, ROWS_PER_NODE, PACK * HIDDEN)
                             ).reshape(R, PACK * HIDDEN)
    h = jnp.maximum(g + c_rep, 0.0)

    t2 = jnp.maximum(
        jnp.dot(h, w2bd_ref[...], preferred_element_type=f32)
        + b2bd_ref[...], 0.0)
    att2 = jnp.maximum(
        jnp.dot(t2, wabd_ref[...], preferred_element_type=f32)
        + babd_ref[...], 0.0)

    m = jnp.max(att2, axis=1, keepdims=True)
    e = jnp.exp(att2 - m)
    li = lax.broadcasted_iota(jnp.int32, (PACK * N_HEADS, PACK * N_HEADS), 0)
    lj = lax.broadcasted_iota(jnp.int32, (PACK * N_HEADS, PACK * N_HEADS), 1)
    s4 = (li // N_HEADS == lj // N_HEADS).astype(f32)
    p2 = e / jnp.dot(e, s4, preferred_element_type=f32)

    p3 = p2.reshape(B, ROWS_PER_NODE, PACK * N_HEADS)
    GL = 2 * PACK * N_HEADS
    el = lax.broadcasted_iota(jnp.int32, (GL, 2 * PACK * OUT_U), 0)
    ec = lax.broadcasted_iota(jnp.int32, (GL, 2 * PACK * OUT_U), 1)
    half = ec // (PACK * OUT_U)
    rloc = (ec % (PACK * OUT_U)) // OUT_U
    ri = lax.broadcasted_iota(jnp.int32, (R, 1), 0) % PACK
    t2d = jnp.concatenate([t2, t2], axis=1)
    outs = []
    for m in range(H // 2):
        pa = p3[:, 4 * m:4 * m + 2, :]
        pb = p3[:, 4 * m + 2:4 * m + 4, :]
        pha = jnp.broadcast_to(pa[:, :, None, :],
                               (B, 2, PACK, PACK * N_HEADS)
                               ).reshape(R, PACK * N_HEADS)
        phb = jnp.broadcast_to(pb[:, :, None, :],
                               (B, 2, PACK, PACK * N_HEADS)
                               ).reshape(R, PACK * N_HEADS)
        php = jnp.concatenate([pha, phb], axis=1)
        pw = jnp.zeros((R, 2 * PACK * OUT_U), dtype=f32)
        for j in range(PACK):
            ej = ((el == 16 * half + PACK * j + rloc)
                  ).astype(f32)
            cand = jnp.dot(php, ej, preferred_element_type=f32)
            pw = pw + jnp.where(ri == j, cand, 0.0)
        term = pw * t2d
        o2 = jnp.sum(term.reshape(B, ROWS_PER_NODE, 2 * PACK * OUT_U),
                     axis=1)
        outs.append(o2[:, 0:U] + o2[:, U:2 * U]
                    + o2[:, 2 * U:3 * U] + o2[:, 3 * U:4 * U])
        outs.append(o2[:, 4 * U:5 * U] + o2[:, 5 * U:6 * U]
                    + o2[:, 6 * U:7 * U] + o2[:, 7 * U:8 * U])
    out_ref[...] = jnp.concatenate(outs, axis=1)


def _dense(g2, c, W2bd, b2bd, Wabd, babd, blk):
    grid = N_NODES // blk
    rows = blk * ROWS_PER_NODE
    return pl.pallas_call(
        functools.partial(_dense_body, blk=blk),
        grid=(grid,),
        in_specs=[
            pl.BlockSpec((rows, PACK * HIDDEN), lambda i: (i, 0)),
            pl.BlockSpec((blk, HIDDEN), lambda i: (i, 0)),
            pl.BlockSpec((PACK * HIDDEN, PACK * OUT_U), lambda i: (0, 0)),
            pl.BlockSpec((1, PACK * OUT_U), lambda i: (0, 0)),
            pl.BlockSpec((PACK * OUT_U, PACK * N_HEADS), lambda i: (0, 0)),
            pl.BlockSpec((1, PACK * N_HEADS), lambda i: (0, 0)),
        ],
        out_specs=pl.BlockSpec((blk, N_HEADS * OUT_U), lambda i: (i, 0)),
        out_shape=jax.ShapeDtypeStruct((N_NODES, N_HEADS * OUT_U),
                                       jnp.float32),
    )(g2, c, W2bd, b2bd, Wabd, babd)


def _block_diag4(w):
    r, c = w.shape
    out = jnp.zeros((PACK * r, PACK * c), dtype=w.dtype)
    for j in range(PACK):
        out = out.at[j * r:(j + 1) * r, j * c:(j + 1) * c].set(w)
    return out


def kernel(x, neighbor_idx, W1, b1, W2, b2, Wa, ba):
    y, c = _precompute(x, W1, b1.reshape(1, HIDDEN))
    idx2d = neighbor_idx.astype(jnp.int32).reshape(N_CHUNKS, CHUNK)
    g = _sc_gather(y, idx2d)
    g2 = g.reshape(N_NODES * K_SAMPLE // PACK, PACK * HIDDEN)
    W2bd = _block_diag4(W2)
    b2bd = jnp.tile(b2, PACK).reshape(1, PACK * OUT_U)
    Wabd = _block_diag4(Wa)
    babd = jnp.tile(ba, PACK).reshape(1, PACK * N_HEADS)
    return _dense(g2, c, W2bd, b2bd, Wabd, babd, blk=200)


# speedup vs baseline: 1.2760x; 1.2760x over previous
import functools

import jax
import jax.numpy as jnp
from jax import lax
from jax.experimental import pallas as pl
from jax.experimental.pallas import tpu as pltpu
from jax.experimental.pallas import tpu_sc as plsc

N_NODES = 10000
K_SAMPLE = 32
D_FEAT = 128
HIDDEN = 32
OUT_U = 16
N_HEADS = 4

CHUNK = 128
N_CHUNKS = (N_NODES * K_SAMPLE) // CHUNK

PACK = 4
ROWS_PER_NODE = K_SAMPLE // PACK


def _precompute_body(x_ref, w1_ref, b1_ref, y_ref, c_ref):
    x = x_ref[...]
    y_ref[...] = jnp.dot(x, w1_ref[0:D_FEAT, :],
                         preferred_element_type=jnp.float32)
    c_ref[...] = jnp.dot(x, w1_ref[D_FEAT:2 * D_FEAT, :],
                         preferred_element_type=jnp.float32) + b1_ref[...]


def _precompute(x, W1, b1):
    return pl.pallas_call(
        _precompute_body,
        out_shape=(
            jax.ShapeDtypeStruct((N_NODES, HIDDEN), jnp.float32),
            jax.ShapeDtypeStruct((N_NODES, HIDDEN), jnp.float32),
        ),
    )(x, W1, b1)


def _sc_gather(y, idx2d):
    n_chunks = idx2d.shape[0]
    info = plsc.get_sparse_core_info()
    nc, ns = info.num_cores, info.num_subcores
    nw = nc * ns
    iters = -(-n_chunks // nw)

    mesh = plsc.VectorSubcoreMesh(core_axis_name="c", subcore_axis_name="s")

    @functools.partial(
        pl.kernel,
        mesh=mesh,
        compiler_params=pltpu.CompilerParams(use_tc_tiling_on_sc=False),
        out_type=jax.ShapeDtypeStruct((n_chunks, CHUNK, HIDDEN),
                                      jnp.float32),
        scratch_types=[
            pltpu.VMEM((CHUNK,), jnp.int32),
            pltpu.VMEM((CHUNK, HIDDEN), jnp.float32),
            pltpu.SemaphoreType.DMA,
        ],
    )
    def k(y_hbm, idx_hbm, out_hbm, idx_v, rows_v, sem):
        wid = lax.axis_index("s") * nc + lax.axis_index("c")

        def body(i, _):
            chunk = i * nw + wid

            @pl.when(chunk < n_chunks)
            def _():
                pltpu.sync_copy(idx_hbm.at[chunk], idx_v)
                pltpu.async_copy(y_hbm.at[idx_v], rows_v, sem).wait()
                pltpu.sync_copy(rows_v, out_hbm.at[chunk])

            return ()

        lax.fori_loop(0, iters, body, ())

    return k(y, idx2d)


def _dense_body(g_ref, c_ref, w2bd_ref, b2bd_ref, wabd_ref, babd_ref,
                out_ref, *, blk):
    B, K, H, U = blk, K_SAMPLE, N_HEADS, OUT_U
    R = B * ROWS_PER_NODE
    f32 = jnp.float32

    g = g_ref[...]
    c = c_ref[...]
    c_rep = jnp.concatenate([c, c, c, c], axis=1)
    c_rep = jnp.broadcast_to(c_rep[:, None, :],
                             (B, ROWS_PER_NODE, PACK * HIDDEN)
                             ).reshape(R, PACK * HIDDEN)
    h = jnp.maximum(g + c_rep, 0.0)

    t2 = jnp.maximum(
        jnp.dot(h, w2bd_ref[...], preferred_element_type=f32)
        + b2bd_ref[...], 0.0)
    att2 = jnp.maximum(
        jnp.dot(t2, wabd_ref[...], preferred_element_type=f32)
        + babd_ref[...], 0.0)

    m = jnp.max(att2, axis=1, keepdims=True)
    e = jnp.exp(att2 - m)
    li = lax.broadcasted_iota(jnp.int32, (PACK * N_HEADS, PACK * N_HEADS), 0)
    lj = lax.broadcasted_iota(jnp.int32, (PACK * N_HEADS, PACK * N_HEADS), 1)
    s4 = (li // N_HEADS == lj // N_HEADS).astype(f32)
    p2 = e / jnp.dot(e, s4, preferred_element_type=f32)

    p3 = p2.reshape(B, ROWS_PER_NODE, PACK * N_HEADS)
    GL = 2 * PACK * N_HEADS
    el = lax.broadcasted_iota(jnp.int32, (GL, 2 * PACK * OUT_U), 0)
    ec = lax.broadcasted_iota(jnp.int32, (GL, 2 * PACK * OUT_U), 1)
    half = ec // (PACK * OUT_U)
    rloc = (ec % (PACK * OUT_U)) // OUT_U
    ri = lax.broadcasted_iota(jnp.int32, (R, 1), 0) % PACK
    t2d = jnp.concatenate([t2, t2], axis=1)
    outs = []
    for m in range(H // 2):
        pa = p3[:, 4 * m:4 * m + 2, :]
        pb = p3[:, 4 * m + 2:4 * m + 4, :]
        pha = jnp.broadcast_to(pa[:, :, None, :],
                               (B, 2, PACK, PACK * N_HEADS)
                               ).reshape(R, PACK * N_HEADS)
        phb = jnp.broadcast_to(pb[:, :, None, :],
                               (B, 2, PACK, PACK * N_HEADS)
                               ).reshape(R, PACK * N_HEADS)
        php = jnp.concatenate([pha, phb], axis=1)
        pw = jnp.zeros((R, 2 * PACK * OUT_U), dtype=f32)
        for j in range(PACK):
            ej = ((el == 16 * half + PACK * j + rloc)
                  ).astype(f32)
            cand = jnp.dot(php, ej, preferred_element_type=f32)
            pw = pw + jnp.where(ri == j, cand, 0.0)
        term = pw * t2d
        o2 = jnp.sum(term.reshape(B, ROWS_PER_NODE, 2 * PACK * OUT_U),
                     axis=1)
        outs.append(o2[:, 0:U] + o2[:, U:2 * U]
                    + o2[:, 2 * U:3 * U] + o2[:, 3 * U:4 * U])
        outs.append(o2[:, 4 * U:5 * U] + o2[:, 5 * U:6 * U]
                    + o2[:, 6 * U:7 * U] + o2[:, 7 * U:8 * U])
    out_ref[...] = jnp.concatenate(outs, axis=1)


def _dense(g2, c, W2bd, b2bd, Wabd, babd, blk, n_nodes):
    grid = n_nodes // blk
    rows = blk * ROWS_PER_NODE
    return pl.pallas_call(
        functools.partial(_dense_body, blk=blk),
        grid=(grid,),
        in_specs=[
            pl.BlockSpec((rows, PACK * HIDDEN), lambda i: (i, 0)),
            pl.BlockSpec((blk, HIDDEN), lambda i: (i, 0)),
            pl.BlockSpec((PACK * HIDDEN, PACK * OUT_U), lambda i: (0, 0)),
            pl.BlockSpec((1, PACK * OUT_U), lambda i: (0, 0)),
            pl.BlockSpec((PACK * OUT_U, PACK * N_HEADS), lambda i: (0, 0)),
            pl.BlockSpec((1, PACK * N_HEADS), lambda i: (0, 0)),
        ],
        out_specs=pl.BlockSpec((blk, N_HEADS * OUT_U), lambda i: (i, 0)),
        out_shape=jax.ShapeDtypeStruct((n_nodes, N_HEADS * OUT_U),
                                       jnp.float32),
    )(g2, c, W2bd, b2bd, Wabd, babd)


def _block_diag4(w):
    r, c = w.shape
    out = jnp.zeros((PACK * r, PACK * c), dtype=w.dtype)
    for j in range(PACK):
        out = out.at[j * r:(j + 1) * r, j * c:(j + 1) * c].set(w)
    return out


N_STAGES = 2


def kernel(x, neighbor_idx, W1, b1, W2, b2, Wa, ba):
    y, c = _precompute(x, W1, b1.reshape(1, HIDDEN))
    idx2d = neighbor_idx.astype(jnp.int32).reshape(N_CHUNKS, CHUNK)
    W2bd = _block_diag4(W2)
    b2bd = jnp.tile(b2, PACK).reshape(1, PACK * OUT_U)
    Wabd = _block_diag4(Wa)
    babd = jnp.tile(ba, PACK).reshape(1, PACK * N_HEADS)
    npn = N_NODES // N_STAGES
    cpn = N_CHUNKS // N_STAGES
    outs = []
    for s in range(N_STAGES):
        g = _sc_gather(y, idx2d[s * cpn:(s + 1) * cpn])
        g2 = g.reshape(npn * K_SAMPLE // PACK, PACK * HIDDEN)
        outs.append(_dense(g2, c[s * npn:(s + 1) * npn],
                           W2bd, b2bd, Wabd, babd, blk=200, n_nodes=npn))
    return jnp.concatenate(outs, axis=0)


# speedup vs baseline: 1.4072x; 1.1028x over previous
import functools

import jax
import jax.numpy as jnp
from jax import lax
from jax.experimental import pallas as pl
from jax.experimental.pallas import tpu as pltpu
from jax.experimental.pallas import tpu_sc as plsc

N_NODES = 10000
K_SAMPLE = 32
D_FEAT = 128
HIDDEN = 32
OUT_U = 16
N_HEADS = 4

CHUNK = 128
N_CHUNKS = (N_NODES * K_SAMPLE) // CHUNK

PACK = 4
ROWS_PER_NODE = K_SAMPLE // PACK


def _precompute_body(x_ref, w1_ref, b1_ref, y_ref, c_ref):
    x = x_ref[...]
    y_ref[...] = jnp.dot(x, w1_ref[0:D_FEAT, :],
                         preferred_element_type=jnp.float32)
    c_ref[...] = jnp.dot(x, w1_ref[D_FEAT:2 * D_FEAT, :],
                         preferred_element_type=jnp.float32) + b1_ref[...]


def _precompute(x, W1, b1):
    return pl.pallas_call(
        _precompute_body,
        out_shape=(
            jax.ShapeDtypeStruct((N_NODES, HIDDEN), jnp.float32),
            jax.ShapeDtypeStruct((N_NODES, HIDDEN), jnp.float32),
        ),
    )(x, W1, b1)


def _sc_gather(y, idx2d):
    n_chunks = idx2d.shape[0]
    info = plsc.get_sparse_core_info()
    nc, ns = info.num_cores, info.num_subcores
    nw = nc * ns
    iters = -(-n_chunks // nw)

    mesh = plsc.VectorSubcoreMesh(core_axis_name="c", subcore_axis_name="s")

    @functools.partial(
        pl.kernel,
        mesh=mesh,
        compiler_params=pltpu.CompilerParams(use_tc_tiling_on_sc=False),
        out_type=jax.ShapeDtypeStruct((n_chunks, CHUNK, HIDDEN),
                                      jnp.float32),
        scratch_types=[
            pltpu.VMEM((CHUNK,), jnp.int32),
            pltpu.VMEM((CHUNK, HIDDEN), jnp.float32),
            pltpu.SemaphoreType.DMA,
        ],
    )
    def k(y_hbm, idx_hbm, out_hbm, idx_v, rows_v, sem):
        wid = lax.axis_index("s") * nc + lax.axis_index("c")

        def body(i, _):
            chunk = i * nw + wid

            @pl.when(chunk < n_chunks)
            def _():
                pltpu.sync_copy(idx_hbm.at[chunk], idx_v)
                pltpu.async_copy(y_hbm.at[idx_v], rows_v, sem).wait()
                pltpu.sync_copy(rows_v, out_hbm.at[chunk])

            return ()

        lax.fori_loop(0, iters, body, ())

    return k(y, idx2d)


def _dense_body(g_ref, c_ref, w2bd_ref, b2bd_ref, wabd_ref, babd_ref,
                out_ref, *, blk):
    B, K, H, U = blk, K_SAMPLE, N_HEADS, OUT_U
    R = B * ROWS_PER_NODE
    f32 = jnp.float32

    g = g_ref[...]
    c = c_ref[...]
    c_rep = jnp.concatenate([c, c, c, c], axis=1)
    c_rep = jnp.broadcast_to(c_rep[:, None, :],
                             (B, ROWS_PER_NODE, PACK * HIDDEN)
                             ).reshape(R, PACK * HIDDEN)
    h = jnp.maximum(g + c_rep, 0.0)

    t2 = jnp.maximum(
        jnp.dot(h, w2bd_ref[...], preferred_element_type=f32)
        + b2bd_ref[...], 0.0)
    att2 = jnp.maximum(
        jnp.dot(t2, wabd_ref[...], preferred_element_type=f32)
        + babd_ref[...], 0.0)

    m = jnp.max(att2, axis=1, keepdims=True)
    e = jnp.exp(att2 - m)
    li = lax.broadcasted_iota(jnp.int32, (PACK * N_HEADS, PACK * N_HEADS), 0)
    lj = lax.broadcasted_iota(jnp.int32, (PACK * N_HEADS, PACK * N_HEADS), 1)
    s4 = (li // N_HEADS == lj // N_HEADS).astype(f32)
    p2 = e / jnp.dot(e, s4, preferred_element_type=f32)

    p3 = p2.reshape(B, ROWS_PER_NODE, PACK * N_HEADS)
    GL = 2 * PACK * N_HEADS
    el = lax.broadcasted_iota(jnp.int32, (GL, 2 * PACK * OUT_U), 0)
    ec = lax.broadcasted_iota(jnp.int32, (GL, 2 * PACK * OUT_U), 1)
    half = ec // (PACK * OUT_U)
    rloc = (ec % (PACK * OUT_U)) // OUT_U
    ri = lax.broadcasted_iota(jnp.int32, (R, 1), 0) % PACK
    t2d = jnp.concatenate([t2, t2], axis=1)
    outs = []
    for m in range(H // 2):
        pa = p3[:, 4 * m:4 * m + 2, :]
        pb = p3[:, 4 * m + 2:4 * m + 4, :]
        pha = jnp.broadcast_to(pa[:, :, None, :],
                               (B, 2, PACK, PACK * N_HEADS)
                               ).reshape(R, PACK * N_HEADS)
        phb = jnp.broadcast_to(pb[:, :, None, :],
                               (B, 2, PACK, PACK * N_HEADS)
                               ).reshape(R, PACK * N_HEADS)
        php = jnp.concatenate([pha, phb], axis=1)
        pw = jnp.zeros((R, 2 * PACK * OUT_U), dtype=f32)
        for j in range(PACK):
            ej = ((el == 16 * half + PACK * j + rloc)
                  ).astype(f32)
            cand = jnp.dot(php, ej, preferred_element_type=f32)
            pw = pw + jnp.where(ri == j, cand, 0.0)
        term = pw * t2d
        o2 = jnp.sum(term.reshape(B, ROWS_PER_NODE, 2 * PACK * OUT_U),
                     axis=1)
        outs.append(o2[:, 0:U] + o2[:, U:2 * U]
                    + o2[:, 2 * U:3 * U] + o2[:, 3 * U:4 * U])
        outs.append(o2[:, 4 * U:5 * U] + o2[:, 5 * U:6 * U]
                    + o2[:, 6 * U:7 * U] + o2[:, 7 * U:8 * U])
    out_ref[...] = jnp.concatenate(outs, axis=1)


def _dense(g2, c, W2bd, b2bd, Wabd, babd, blk, n_nodes):
    grid = n_nodes // blk
    rows = blk * ROWS_PER_NODE
    return pl.pallas_call(
        functools.partial(_dense_body, blk=blk),
        grid=(grid,),
        in_specs=[
            pl.BlockSpec((rows, PACK * HIDDEN), lambda i: (i, 0)),
            pl.BlockSpec((blk, HIDDEN), lambda i: (i, 0)),
            pl.BlockSpec((PACK * HIDDEN, PACK * OUT_U), lambda i: (0, 0)),
            pl.BlockSpec((1, PACK * OUT_U), lambda i: (0, 0)),
            pl.BlockSpec((PACK * OUT_U, PACK * N_HEADS), lambda i: (0, 0)),
            pl.BlockSpec((1, PACK * N_HEADS), lambda i: (0, 0)),
        ],
        out_specs=pl.BlockSpec((blk, N_HEADS * OUT_U), lambda i: (i, 0)),
        out_shape=jax.ShapeDtypeStruct((n_nodes, N_HEADS * OUT_U),
                                       jnp.float32),
    )(g2, c, W2bd, b2bd, Wabd, babd)


def _block_diag4(w):
    r, c = w.shape
    out = jnp.zeros((PACK * r, PACK * c), dtype=w.dtype)
    for j in range(PACK):
        out = out.at[j * r:(j + 1) * r, j * c:(j + 1) * c].set(w)
    return out


N_STAGES = 5


def kernel(x, neighbor_idx, W1, b1, W2, b2, Wa, ba):
    y, c = _precompute(x, W1, b1.reshape(1, HIDDEN))
    idx2d = neighbor_idx.astype(jnp.int32).reshape(N_CHUNKS, CHUNK)
    W2bd = _block_diag4(W2)
    b2bd = jnp.tile(b2, PACK).reshape(1, PACK * OUT_U)
    Wabd = _block_diag4(Wa)
    babd = jnp.tile(ba, PACK).reshape(1, PACK * N_HEADS)
    npn = N_NODES // N_STAGES
    cpn = N_CHUNKS // N_STAGES
    outs = []
    for s in range(N_STAGES):
        g = _sc_gather(y, idx2d[s * cpn:(s + 1) * cpn])
        g2 = g.reshape(npn * K_SAMPLE // PACK, PACK * HIDDEN)
        outs.append(_dense(g2, c[s * npn:(s + 1) * npn],
                           W2bd, b2bd, Wabd, babd, blk=200, n_nodes=npn))
    return jnp.concatenate(outs, axis=0)


# speedup vs baseline: 1.5551x; 1.1051x over previous
import functools

import jax
import jax.numpy as jnp
from jax import lax
from jax.experimental import pallas as pl
from jax.experimental.pallas import tpu as pltpu
from jax.experimental.pallas import tpu_sc as plsc

N_NODES = 10000
K_SAMPLE = 32
D_FEAT = 128
HIDDEN = 32
OUT_U = 16
N_HEADS = 4

CHUNK = 128
N_CHUNKS = (N_NODES * K_SAMPLE) // CHUNK

PACK = 4
ROWS_PER_NODE = K_SAMPLE // PACK


def _precompute_body(x_ref, w1_ref, b1_ref, y_ref, c_ref):
    x = x_ref[...]
    y_ref[...] = jnp.dot(x, w1_ref[0:D_FEAT, :],
                         preferred_element_type=jnp.float32)
    c_ref[...] = jnp.dot(x, w1_ref[D_FEAT:2 * D_FEAT, :],
                         preferred_element_type=jnp.float32) + b1_ref[...]


def _precompute(x, W1, b1):
    return pl.pallas_call(
        _precompute_body,
        out_shape=(
            jax.ShapeDtypeStruct((N_NODES, HIDDEN), jnp.float32),
            jax.ShapeDtypeStruct((N_NODES, HIDDEN), jnp.float32),
        ),
    )(x, W1, b1)


def _sc_gather(y, idx2d):
    n_chunks = idx2d.shape[0]
    info = plsc.get_sparse_core_info()
    nc, ns = info.num_cores, info.num_subcores
    nw = nc * ns
    iters = -(-n_chunks // nw)

    mesh = plsc.VectorSubcoreMesh(core_axis_name="c", subcore_axis_name="s")

    @functools.partial(
        pl.kernel,
        mesh=mesh,
        compiler_params=pltpu.CompilerParams(use_tc_tiling_on_sc=False),
        out_type=jax.ShapeDtypeStruct((n_chunks, CHUNK, HIDDEN),
                                      jnp.float32),
        scratch_types=[
            pltpu.VMEM((CHUNK,), jnp.int32),
            pltpu.VMEM((CHUNK, HIDDEN), jnp.float32),
            pltpu.SemaphoreType.DMA,
        ],
    )
    def k(y_hbm, idx_hbm, out_hbm, idx_v, rows_v, sem):
        wid = lax.axis_index("s") * nc + lax.axis_index("c")

        def body(i, _):
            chunk = i * nw + wid

            @pl.when(chunk < n_chunks)
            def _():
                pltpu.sync_copy(idx_hbm.at[chunk], idx_v)
                pltpu.async_copy(y_hbm.at[idx_v], rows_v, sem).wait()
                pltpu.sync_copy(rows_v, out_hbm.at[chunk])

            return ()

        lax.fori_loop(0, iters, body, ())

    return k(y, idx2d)


def _dense_body(g_ref, c_ref, w2bd_ref, b2bd_ref, wabd_ref, babd_ref,
                sseg_ref, out_ref, *, blk):
    B, K, H, U = blk, K_SAMPLE, N_HEADS, OUT_U
    R = B * ROWS_PER_NODE
    f32 = jnp.float32

    g = g_ref[...]
    c = c_ref[...]
    c_rep = jnp.concatenate([c, c, c, c], axis=1)
    c_rep = jnp.broadcast_to(c_rep[:, None, :],
                             (B, ROWS_PER_NODE, PACK * HIDDEN)
                             ).reshape(R, PACK * HIDDEN)
    h = jnp.maximum(g + c_rep, 0.0)

    t2 = jnp.maximum(
        jnp.dot(h, w2bd_ref[...], preferred_element_type=f32)
        + b2bd_ref[...], 0.0)
    att2 = jnp.maximum(
        jnp.dot(t2, wabd_ref[...], preferred_element_type=f32)
        + babd_ref[...], 0.0)

    m = jnp.max(att2, axis=1, keepdims=True)
    e = jnp.exp(att2 - m)
    li = lax.broadcasted_iota(jnp.int32, (PACK * N_HEADS, PACK * N_HEADS), 0)
    lj = lax.broadcasted_iota(jnp.int32, (PACK * N_HEADS, PACK * N_HEADS), 1)
    s4 = (li // N_HEADS == lj // N_HEADS).astype(f32)
    p2 = e / jnp.dot(e, s4, preferred_element_type=f32)

    p3 = p2.reshape(B, ROWS_PER_NODE, PACK * N_HEADS)
    GL = 2 * PACK * N_HEADS
    el = lax.broadcasted_iota(jnp.int32, (GL, 2 * PACK * OUT_U), 0)
    ec = lax.broadcasted_iota(jnp.int32, (GL, 2 * PACK * OUT_U), 1)
    half = ec // (PACK * OUT_U)
    rloc = (ec % (PACK * OUT_U)) // OUT_U
    ri = lax.broadcasted_iota(jnp.int32, (R, 1), 0) % PACK
    fl = lax.broadcasted_iota(jnp.int32, (2 * PACK * OUT_U, 2 * OUT_U), 0)
    fc = lax.broadcasted_iota(jnp.int32, (2 * PACK * OUT_U, 2 * OUT_U), 1)
    fold = ((fl % OUT_U == fc % OUT_U)
            & (fl // (PACK * OUT_U) == fc // OUT_U)).astype(f32)
    t2d = jnp.concatenate([t2, t2], axis=1)
    outs = []
    for m in range(H // 2):
        pa = p3[:, 4 * m:4 * m + 2, :]
        pb = p3[:, 4 * m + 2:4 * m + 4, :]
        pha = jnp.broadcast_to(pa[:, :, None, :],
                               (B, 2, PACK, PACK * N_HEADS)
                               ).reshape(R, PACK * N_HEADS)
        phb = jnp.broadcast_to(pb[:, :, None, :],
                               (B, 2, PACK, PACK * N_HEADS)
                               ).reshape(R, PACK * N_HEADS)
        php = jnp.concatenate([pha, phb], axis=1)
        pw = jnp.zeros((R, 2 * PACK * OUT_U), dtype=f32)
        for j in range(PACK):
            ej = ((el == 16 * half + PACK * j + rloc)
                  ).astype(f32)
            cand = jnp.dot(php, ej, preferred_element_type=f32)
            pw = pw + jnp.where(ri == j, cand, 0.0)
        term = pw * t2d
        tmp = jnp.dot(term, fold, preferred_element_type=f32)
        outs.append(jnp.dot(sseg_ref[...], tmp,
                            preferred_element_type=f32))
    out_ref[...] = jnp.concatenate(outs, axis=1)


def _dense(g2, c, W2bd, b2bd, Wabd, babd, sseg, blk, n_nodes):
    grid = n_nodes // blk
    rows = blk * ROWS_PER_NODE
    return pl.pallas_call(
        functools.partial(_dense_body, blk=blk),
        grid=(grid,),
        in_specs=[
            pl.BlockSpec((rows, PACK * HIDDEN), lambda i: (i, 0)),
            pl.BlockSpec((blk, HIDDEN), lambda i: (i, 0)),
            pl.BlockSpec((PACK * HIDDEN, PACK * OUT_U), lambda i: (0, 0)),
            pl.BlockSpec((1, PACK * OUT_U), lambda i: (0, 0)),
            pl.BlockSpec((PACK * OUT_U, PACK * N_HEADS), lambda i: (0, 0)),
            pl.BlockSpec((1, PACK * N_HEADS), lambda i: (0, 0)),
            pl.BlockSpec((blk, blk * ROWS_PER_NODE), lambda i: (0, 0)),
        ],
        out_specs=pl.BlockSpec((blk, N_HEADS * OUT_U), lambda i: (i, 0)),
        out_shape=jax.ShapeDtypeStruct((n_nodes, N_HEADS * OUT_U),
                                       jnp.float32),
    )(g2, c, W2bd, b2bd, Wabd, babd, sseg)


def _block_diag4(w):
    r, c = w.shape
    out = jnp.zeros((PACK * r, PACK * c), dtype=w.dtype)
    for j in range(PACK):
        out = out.at[j * r:(j + 1) * r, j * c:(j + 1) * c].set(w)
    return out


N_STAGES = 5


def kernel(x, neighbor_idx, W1, b1, W2, b2, Wa, ba):
    y, c = _precompute(x, W1, b1.reshape(1, HIDDEN))
    idx2d = neighbor_idx.astype(jnp.int32).reshape(N_CHUNKS, CHUNK)
    W2bd = _block_diag4(W2)
    b2bd = jnp.tile(b2, PACK).reshape(1, PACK * OUT_U)
    Wabd = _block_diag4(Wa)
    babd = jnp.tile(ba, PACK).reshape(1, PACK * N_HEADS)
    npn = N_NODES // N_STAGES
    cpn = N_CHUNKS // N_STAGES
    blk = 200
    ii = jnp.arange(blk)[:, None]
    jj = jnp.arange(blk * ROWS_PER_NODE)[None, :]
    sseg = (jj // ROWS_PER_NODE == ii).astype(jnp.float32)
    outs = []
    for s in range(N_STAGES):
        g = _sc_gather(y, idx2d[s * cpn:(s + 1) * cpn])
        g2 = g.reshape(npn * K_SAMPLE // PACK, PACK * HIDDEN)
        outs.append(_dense(g2, c[s * npn:(s + 1) * npn],
                           W2bd, b2bd, Wabd, babd, sseg, blk=blk,
                           n_nodes=npn))
    return jnp.concatenate(outs, axis=0)


# speedup vs baseline: 1.5978x; 1.0274x over previous
import functools

import jax
import jax.numpy as jnp
from jax import lax
from jax.experimental import pallas as pl
from jax.experimental.pallas import tpu as pltpu
from jax.experimental.pallas import tpu_sc as plsc

N_NODES = 10000
K_SAMPLE = 32
D_FEAT = 128
HIDDEN = 32
OUT_U = 16
N_HEADS = 4

CHUNK = 128
N_CHUNKS = (N_NODES * K_SAMPLE) // CHUNK

PACK = 4
ROWS_PER_NODE = K_SAMPLE // PACK


def _precompute_body(x_ref, w1_ref, b1_ref, y_ref, c_ref):
    x = x_ref[...]
    y_ref[...] = jnp.dot(x, w1_ref[0:D_FEAT, :],
                         preferred_element_type=jnp.float32)
    c_ref[...] = jnp.dot(x, w1_ref[D_FEAT:2 * D_FEAT, :],
                         preferred_element_type=jnp.float32) + b1_ref[...]


def _precompute(x, W1, b1):
    return pl.pallas_call(
        _precompute_body,
        out_shape=(
            jax.ShapeDtypeStruct((N_NODES, HIDDEN), jnp.float32),
            jax.ShapeDtypeStruct((N_NODES, HIDDEN), jnp.float32),
        ),
    )(x, W1, b1)


def _sc_gather(y, idx2d):
    n_chunks = idx2d.shape[0]
    info = plsc.get_sparse_core_info()
    nc, ns = info.num_cores, info.num_subcores
    nw = nc * ns
    iters = -(-n_chunks // nw)

    mesh = plsc.VectorSubcoreMesh(core_axis_name="c", subcore_axis_name="s")

    @functools.partial(
        pl.kernel,
        mesh=mesh,
        compiler_params=pltpu.CompilerParams(use_tc_tiling_on_sc=False),
        out_type=jax.ShapeDtypeStruct((n_chunks, CHUNK, HIDDEN),
                                      jnp.float32),
        scratch_types=[
            pltpu.VMEM((CHUNK,), jnp.int32),
            pltpu.VMEM((CHUNK,), jnp.int32),
            pltpu.VMEM((CHUNK, HIDDEN), jnp.float32),
            pltpu.VMEM((CHUNK, HIDDEN), jnp.float32),
            pltpu.SemaphoreType.DMA,
            pltpu.SemaphoreType.DMA,
        ],
    )
    def k(y_hbm, idx_hbm, out_hbm, idx_a, idx_b, rows_a, rows_b, sa, sb):
        wid = lax.axis_index("s") * nc + lax.axis_index("c")

        def body(i, _):
            ca = (2 * i) * nw + wid
            cb = (2 * i + 1) * nw + wid

            @pl.when(ca < n_chunks)
            def _():
                pltpu.sync_copy(idx_hbm.at[ca], idx_a)
                pltpu.async_copy(y_hbm.at[idx_a], rows_a, sa)

            @pl.when(cb < n_chunks)
            def _():
                pltpu.sync_copy(idx_hbm.at[cb], idx_b)
                pltpu.async_copy(y_hbm.at[idx_b], rows_b, sb)

            @pl.when(ca < n_chunks)
            def _():
                pltpu.make_async_copy(y_hbm.at[idx_a], rows_a, sa).wait()
                pltpu.sync_copy(rows_a, out_hbm.at[ca])

            @pl.when(cb < n_chunks)
            def _():
                pltpu.make_async_copy(y_hbm.at[idx_b], rows_b, sb).wait()
                pltpu.sync_copy(rows_b, out_hbm.at[cb])

            return ()

        lax.fori_loop(0, -(-iters // 2), body, ())

    return k(y, idx2d)


def _dense_body(g_ref, c_ref, w2bd_ref, b2bd_ref, wabd_ref, babd_ref,
                sseg_ref, out_ref, *, blk):
    B, K, H, U = blk, K_SAMPLE, N_HEADS, OUT_U
    R = B * ROWS_PER_NODE
    f32 = jnp.float32

    g = g_ref[...]
    c = c_ref[...]
    c_rep = jnp.concatenate([c, c, c, c], axis=1)
    c_rep = jnp.broadcast_to(c_rep[:, None, :],
                             (B, ROWS_PER_NODE, PACK * HIDDEN)
                             ).reshape(R, PACK * HIDDEN)
    h = jnp.maximum(g + c_rep, 0.0)

    t2 = jnp.maximum(
        jnp.dot(h, w2bd_ref[...], preferred_element_type=f32)
        + b2bd_ref[...], 0.0)
    att2 = jnp.maximum(
        jnp.dot(t2, wabd_ref[...], preferred_element_type=f32)
        + babd_ref[...], 0.0)

    m = jnp.max(att2, axis=1, keepdims=True)
    e = jnp.exp(att2 - m)
    li = lax.broadcasted_iota(jnp.int32, (PACK * N_HEADS, PACK * N_HEADS), 0)
    lj = lax.broadcasted_iota(jnp.int32, (PACK * N_HEADS, PACK * N_HEADS), 1)
    s4 = (li // N_HEADS == lj // N_HEADS).astype(f32)
    p2 = e / jnp.dot(e, s4, preferred_element_type=f32)

    p3 = p2.reshape(B, ROWS_PER_NODE, PACK * N_HEADS)
    GL = 2 * PACK * N_HEADS
    el = lax.broadcasted_iota(jnp.int32, (GL, 2 * PACK * OUT_U), 0)
    ec = lax.broadcasted_iota(jnp.int32, (GL, 2 * PACK * OUT_U), 1)
    half = ec // (PACK * OUT_U)
    rloc = (ec % (PACK * OUT_U)) // OUT_U
    fl = lax.broadcasted_iota(jnp.int32, (2 * PACK * OUT_U, 2 * OUT_U), 0)
    fc = lax.broadcasted_iota(jnp.int32, (2 * PACK * OUT_U, 2 * OUT_U), 1)
    fold = ((fl % OUT_U == fc % OUT_U)
            & (fl // (PACK * OUT_U) == fc // OUT_U)).astype(f32)
    ri = lax.broadcasted_iota(jnp.int32, (R, 1), 0) % PACK
    t2d = jnp.concatenate([t2, t2], axis=1)
    outs = []
    for m in range(H // 2):
        pa = p3[:, 4 * m:4 * m + 2, :]
        pb = p3[:, 4 * m + 2:4 * m + 4, :]
        pha = jnp.broadcast_to(pa[:, :, None, :],
                               (B, 2, PACK, PACK * N_HEADS)
                               ).reshape(R, PACK * N_HEADS)
        phb = jnp.broadcast_to(pb[:, :, None, :],
                               (B, 2, PACK, PACK * N_HEADS)
                               ).reshape(R, PACK * N_HEADS)
        php = jnp.concatenate([pha, phb], axis=1)
        pw = jnp.zeros((R, 2 * PACK * OUT_U), dtype=f32)
        for j in range(PACK):
            ej = ((el == 16 * half + PACK * j + rloc)
                  ).astype(f32)
            cand = jnp.dot(php, ej, preferred_element_type=f32)
            pw = pw + jnp.where(ri == j, cand, 0.0)
        term = pw * t2d
        tmp = jnp.dot(term, fold, preferred_element_type=f32)
        outs.append(jnp.dot(sseg_ref[...], tmp,
                            preferred_element_type=f32))
    out_ref[...] = jnp.concatenate(outs, axis=1)


def _dense(g2, c, W2bd, b2bd, Wabd, babd, sseg, blk, n_nodes):
    grid = n_nodes // blk
    rows = blk * ROWS_PER_NODE
    return pl.pallas_call(
        functools.partial(_dense_body, blk=blk),
        grid=(grid,),
        in_specs=[
            pl.BlockSpec((rows, PACK * HIDDEN), lambda i: (i, 0)),
            pl.BlockSpec((blk, HIDDEN), lambda i: (i, 0)),
            pl.BlockSpec((PACK * HIDDEN, PACK * OUT_U), lambda i: (0, 0)),
            pl.BlockSpec((1, PACK * OUT_U), lambda i: (0, 0)),
            pl.BlockSpec((PACK * OUT_U, PACK * N_HEADS), lambda i: (0, 0)),
            pl.BlockSpec((1, PACK * N_HEADS), lambda i: (0, 0)),
            pl.BlockSpec((blk, blk * ROWS_PER_NODE), lambda i: (0, 0)),
        ],
        out_specs=pl.BlockSpec((blk, N_HEADS * OUT_U), lambda i: (i, 0)),
        out_shape=jax.ShapeDtypeStruct((n_nodes, N_HEADS * OUT_U),
                                       jnp.float32),
    )(g2, c, W2bd, b2bd, Wabd, babd, sseg)


def _block_diag4(w):
    r, c = w.shape
    out = jnp.zeros((PACK * r, PACK * c), dtype=w.dtype)
    for j in range(PACK):
        out = out.at[j * r:(j + 1) * r, j * c:(j + 1) * c].set(w)
    return out


N_STAGES = 5


def kernel(x, neighbor_idx, W1, b1, W2, b2, Wa, ba):
    y, c = _precompute(x, W1, b1.reshape(1, HIDDEN))
    idx2d = neighbor_idx.astype(jnp.int32).reshape(N_CHUNKS, CHUNK)
    W2bd = _block_diag4(W2)
    b2bd = jnp.tile(b2, PACK).reshape(1, PACK * OUT_U)
    Wabd = _block_diag4(Wa)
    babd = jnp.tile(ba, PACK).reshape(1, PACK * N_HEADS)
    npn = N_NODES // N_STAGES
    cpn = N_CHUNKS // N_STAGES
    blk = 200
    ii = jnp.arange(blk)[:, None]
    jj = jnp.arange(blk * ROWS_PER_NODE)[None, :]
    sseg = (jj // ROWS_PER_NODE == ii).astype(jnp.float32)
    outs = []
    for s in range(N_STAGES):
        g = _sc_gather(y, idx2d[s * cpn:(s + 1) * cpn])
        g2 = g.reshape(npn * K_SAMPLE // PACK, PACK * HIDDEN)
        outs.append(_dense(g2, c[s * npn:(s + 1) * npn],
                           W2bd, b2bd, Wabd, babd, sseg, blk=blk,
                           n_nodes=npn))
    return jnp.concatenate(outs, axis=0)


# speedup vs baseline: 1.6259x; 1.0176x over previous
import functools

import jax
import jax.numpy as jnp
from jax import lax
from jax.experimental import pallas as pl
from jax.experimental.pallas import tpu as pltpu
from jax.experimental.pallas import tpu_sc as plsc

N_NODES = 10000
K_SAMPLE = 32
D_FEAT = 128
HIDDEN = 32
OUT_U = 16
N_HEADS = 4

CHUNK = 128
N_CHUNKS = (N_NODES * K_SAMPLE) // CHUNK

PACK = 4
ROWS_PER_NODE = K_SAMPLE // PACK


def _precompute_body(x_ref, w1_ref, b1_ref, y_ref, c_ref):
    x = x_ref[...]
    y_ref[...] = jnp.dot(x, w1_ref[0:D_FEAT, :],
                         preferred_element_type=jnp.float32)
    c_ref[...] = jnp.dot(x, w1_ref[D_FEAT:2 * D_FEAT, :],
                         preferred_element_type=jnp.float32) + b1_ref[...]


def _precompute(x, W1, b1):
    return pl.pallas_call(
        _precompute_body,
        out_shape=(
            jax.ShapeDtypeStruct((N_NODES, HIDDEN), jnp.float32),
            jax.ShapeDtypeStruct((N_NODES, HIDDEN), jnp.float32),
        ),
    )(x, W1, b1)


def _sc_gather(y, idx2d):
    n_chunks = idx2d.shape[0]
    info = plsc.get_sparse_core_info()
    nc, ns = info.num_cores, info.num_subcores
    nw = nc * ns
    iters = -(-n_chunks // nw)

    mesh = plsc.VectorSubcoreMesh(core_axis_name="c", subcore_axis_name="s")

    @functools.partial(
        pl.kernel,
        mesh=mesh,
        compiler_params=pltpu.CompilerParams(use_tc_tiling_on_sc=False),
        out_type=jax.ShapeDtypeStruct((n_chunks, CHUNK, HIDDEN),
                                      jnp.float32),
        scratch_types=[
            pltpu.VMEM((4, CHUNK), jnp.int32),
            pltpu.VMEM((4, CHUNK, HIDDEN), jnp.float32),
            pltpu.SemaphoreType.DMA,
            pltpu.SemaphoreType.DMA,
            pltpu.SemaphoreType.DMA,
            pltpu.SemaphoreType.DMA,
        ],
    )
    def k(y_hbm, idx_hbm, out_hbm, idx_v, rows_v, s0, s1, s2, s3):
        wid = lax.axis_index("s") * nc + lax.axis_index("c")
        sems = (s0, s1, s2, s3)

        def body(i, _):
            cs = [(4 * i + b) * nw + wid for b in range(4)]
            for b in range(4):
                @pl.when(cs[b] < n_chunks)
                def _(b=b):
                    pltpu.sync_copy(idx_hbm.at[cs[b]], idx_v.at[b])
                    pltpu.async_copy(y_hbm.at[idx_v.at[b]], rows_v.at[b],
                                     sems[b])

            for b in range(4):
                @pl.when(cs[b] < n_chunks)
                def _(b=b):
                    pltpu.make_async_copy(y_hbm.at[idx_v.at[b]],
                                          rows_v.at[b], sems[b]).wait()
                    pltpu.sync_copy(rows_v.at[b], out_hbm.at[cs[b]])

            return ()

        lax.fori_loop(0, -(-iters // 4), body, ())

    return k(y, idx2d)


def _dense_body(g_ref, c_ref, w2bd_ref, b2bd_ref, wabd_ref, babd_ref,
                sseg_ref, out_ref, *, blk):
    B, K, H, U = blk, K_SAMPLE, N_HEADS, OUT_U
    R = B * ROWS_PER_NODE
    f32 = jnp.float32

    g = g_ref[...]
    c = c_ref[...]
    c_rep = jnp.concatenate([c, c, c, c], axis=1)
    c_rep = jnp.broadcast_to(c_rep[:, None, :],
                             (B, ROWS_PER_NODE, PACK * HIDDEN)
                             ).reshape(R, PACK * HIDDEN)
    h = jnp.maximum(g + c_rep, 0.0)

    t2 = jnp.maximum(
        jnp.dot(h, w2bd_ref[...], preferred_element_type=f32)
        + b2bd_ref[...], 0.0)
    att2 = jnp.maximum(
        jnp.dot(t2, wabd_ref[...], preferred_element_type=f32)
        + babd_ref[...], 0.0)

    m = jnp.max(att2, axis=1, keepdims=True)
    e = jnp.exp(att2 - m)
    li = lax.broadcasted_iota(jnp.int32, (PACK * N_HEADS, PACK * N_HEADS), 0)
    lj = lax.broadcasted_iota(jnp.int32, (PACK * N_HEADS, PACK * N_HEADS), 1)
    s4 = (li // N_HEADS == lj // N_HEADS).astype(f32)
    p2 = e / jnp.dot(e, s4, preferred_element_type=f32)

    p3 = p2.reshape(B, ROWS_PER_NODE, PACK * N_HEADS)
    GL = 2 * PACK * N_HEADS
    el = lax.broadcasted_iota(jnp.int32, (GL, 2 * PACK * OUT_U), 0)
    ec = lax.broadcasted_iota(jnp.int32, (GL, 2 * PACK * OUT_U), 1)
    half = ec // (PACK * OUT_U)
    rloc = (ec % (PACK * OUT_U)) // OUT_U
    fl = lax.broadcasted_iota(jnp.int32, (2 * PACK * OUT_U, 2 * OUT_U), 0)
    fc = lax.broadcasted_iota(jnp.int32, (2 * PACK * OUT_U, 2 * OUT_U), 1)
    fold = ((fl % OUT_U == fc % OUT_U)
            & (fl // (PACK * OUT_U) == fc // OUT_U)).astype(f32)
    ri = lax.broadcasted_iota(jnp.int32, (R, 1), 0) % PACK
    t2d = jnp.concatenate([t2, t2], axis=1)
    outs = []
    for m in range(H // 2):
        pa = p3[:, 4 * m:4 * m + 2, :]
        pb = p3[:, 4 * m + 2:4 * m + 4, :]
        pha = jnp.broadcast_to(pa[:, :, None, :],
                               (B, 2, PACK, PACK * N_HEADS)
                               ).reshape(R, PACK * N_HEADS)
        phb = jnp.broadcast_to(pb[:, :, None, :],
                               (B, 2, PACK, PACK * N_HEADS)
                               ).reshape(R, PACK * N_HEADS)
        php = jnp.concatenate([pha, phb], axis=1)
        pw = jnp.zeros((R, 2 * PACK * OUT_U), dtype=f32)
        for j in range(PACK):
            ej = ((el == 16 * half + PACK * j + rloc)
                  ).astype(f32)
            cand = jnp.dot(php, ej, preferred_element_type=f32)
            pw = pw + jnp.where(ri == j, cand, 0.0)
        term = pw * t2d
        tmp = jnp.dot(term, fold, preferred_element_type=f32)
        outs.append(jnp.dot(sseg_ref[...], tmp,
                            preferred_element_type=f32))
    out_ref[...] = jnp.concatenate(outs, axis=1)


def _dense(g2, c, W2bd, b2bd, Wabd, babd, sseg, blk, n_nodes):
    grid = n_nodes // blk
    rows = blk * ROWS_PER_NODE
    return pl.pallas_call(
        functools.partial(_dense_body, blk=blk),
        grid=(grid,),
        in_specs=[
            pl.BlockSpec((rows, PACK * HIDDEN), lambda i: (i, 0)),
            pl.BlockSpec((blk, HIDDEN), lambda i: (i, 0)),
            pl.BlockSpec((PACK * HIDDEN, PACK * OUT_U), lambda i: (0, 0)),
            pl.BlockSpec((1, PACK * OUT_U), lambda i: (0, 0)),
            pl.BlockSpec((PACK * OUT_U, PACK * N_HEADS), lambda i: (0, 0)),
            pl.BlockSpec((1, PACK * N_HEADS), lambda i: (0, 0)),
            pl.BlockSpec((blk, blk * ROWS_PER_NODE), lambda i: (0, 0)),
        ],
        out_specs=pl.BlockSpec((blk, N_HEADS * OUT_U), lambda i: (i, 0)),
        out_shape=jax.ShapeDtypeStruct((n_nodes, N_HEADS * OUT_U),
                                       jnp.float32),
    )(g2, c, W2bd, b2bd, Wabd, babd, sseg)


def _block_diag4(w):
    r, c = w.shape
    out = jnp.zeros((PACK * r, PACK * c), dtype=w.dtype)
    for j in range(PACK):
        out = out.at[j * r:(j + 1) * r, j * c:(j + 1) * c].set(w)
    return out


N_STAGES = 5


def kernel(x, neighbor_idx, W1, b1, W2, b2, Wa, ba):
    y, c = _precompute(x, W1, b1.reshape(1, HIDDEN))
    idx2d = neighbor_idx.astype(jnp.int32).reshape(N_CHUNKS, CHUNK)
    W2bd = _block_diag4(W2)
    b2bd = jnp.tile(b2, PACK).reshape(1, PACK * OUT_U)
    Wabd = _block_diag4(Wa)
    babd = jnp.tile(ba, PACK).reshape(1, PACK * N_HEADS)
    npn = N_NODES // N_STAGES
    cpn = N_CHUNKS // N_STAGES
    blk = 200
    ii = jnp.arange(blk)[:, None]
    jj = jnp.arange(blk * ROWS_PER_NODE)[None, :]
    sseg = (jj // ROWS_PER_NODE == ii).astype(jnp.float32)
    outs = []
    for s in range(N_STAGES):
        g = _sc_gather(y, idx2d[s * cpn:(s + 1) * cpn])
        g2 = g.reshape(npn * K_SAMPLE // PACK, PACK * HIDDEN)
        outs.append(_dense(g2, c[s * npn:(s + 1) * npn],
                           W2bd, b2bd, Wabd, babd, sseg, blk=blk,
                           n_nodes=npn))
    return jnp.concatenate(outs, axis=0)
